# Initial kernel scaffold; baseline (speedup 1.0000x reference)
import jax
import jax.numpy as jnp
from jax.experimental import pallas as pl


def kernel(z_in, adj, W2, W3):
    raise NotImplementedError("write your pallas kernel here")


# speedup vs baseline: 1.6296x; 1.6296x over previous
import jax
import jax.numpy as jnp
from jax.experimental import pallas as pl
from jax.experimental.pallas import tpu as pltpu

_N = 4096
_D1 = 256
_D2 = 128
_TM = 512


def _encode_body(adj_ref, z_ref, w_ref, mu_ref, logvar_ref, h_ref):
    @pl.when(pl.program_id(0) == 0)
    def _():
        h_ref[...] = jnp.dot(
            z_ref[...], w_ref[...], preferred_element_type=jnp.float32
        )

    muvar = jnp.dot(adj_ref[...], h_ref[...], preferred_element_type=jnp.float32)
    mu_ref[...] = muvar[:, :_D2]
    logvar_ref[...] = muvar[:, _D2:]


def _decode_body(mu_blk_ref, mu_all_ref, out_ref):
    out_ref[...] = jax.lax.dot_general(
        mu_blk_ref[...],
        mu_all_ref[...],
        dimension_numbers=(((1,), (1,)), ((), ())),
        preferred_element_type=jnp.float32,
    )


def kernel(z_in, adj, W2, W3):
    wcat = jnp.concatenate([W2, W3], axis=1)

    mu, logvar = pl.pallas_call(
        _encode_body,
        grid=(_N // _TM,),
        in_specs=[
            pl.BlockSpec((_TM, _N), lambda i: (i, 0)),
            pl.BlockSpec((_N, _D1), lambda i: (0, 0)),
            pl.BlockSpec((_D1, 2 * _D2), lambda i: (0, 0)),
        ],
        out_specs=[
            pl.BlockSpec((_TM, _D2), lambda i: (i, 0)),
            pl.BlockSpec((_TM, _D2), lambda i: (i, 0)),
        ],
        out_shape=[
            jax.ShapeDtypeStruct((_N, _D2), jnp.float32),
            jax.ShapeDtypeStruct((_N, _D2), jnp.float32),
        ],
        scratch_shapes=[pltpu.VMEM((_N, 2 * _D2), jnp.float32)],
    )(adj, z_in, wcat)

    adj_recon = pl.pallas_call(
        _decode_body,
        grid=(_N // _TM,),
        in_specs=[
            pl.BlockSpec((_TM, _D2), lambda i: (i, 0)),
            pl.BlockSpec((_N, _D2), lambda i: (0, 0)),
        ],
        out_specs=pl.BlockSpec((_TM, _N), lambda i: (i, 0)),
        out_shape=jax.ShapeDtypeStruct((_N, _N), jnp.float32),
    )(mu, mu)

    return (adj_recon, mu, logvar, mu)


# speedup vs baseline: 1.6590x; 1.0180x over previous
import jax
import jax.numpy as jnp
from jax.experimental import pallas as pl
from jax.experimental.pallas import tpu as pltpu

_N = 4096
_D1 = 256
_D2 = 128
_TM = 512
_G = _N // _TM


def _body(adj_ref, z_ref, w_ref, recon_ref, mu_ref, logvar_ref, h_ref):
    i = pl.program_id(0)

    @pl.when(i == 0)
    def _():
        h_ref[...] = jnp.dot(
            z_ref[...], w_ref[...], preferred_element_type=jnp.float32
        )

    @pl.when(i < _G)
    def _():
        muvar = jnp.dot(
            adj_ref[...], h_ref[...], preferred_element_type=jnp.float32
        )
        r0 = i * _TM
        mu_ref[pl.ds(r0, _TM), :] = muvar[:, :_D2]
        logvar_ref[pl.ds(r0, _TM), :] = muvar[:, _D2:]

    @pl.when(i >= _G)
    def _():
        r0 = (i - _G) * _TM
        blk = mu_ref[pl.ds(r0, _TM), :]
        recon_ref[...] = jax.lax.dot_general(
            blk,
            mu_ref[...],
            dimension_numbers=(((1,), (1,)), ((), ())),
            preferred_element_type=jnp.float32,
        )


def kernel(z_in, adj, W2, W3):
    wcat = jnp.concatenate([W2, W3], axis=1)

    adj_recon, mu, logvar = pl.pallas_call(
        _body,
        grid=(2 * _G,),
        in_specs=[
            pl.BlockSpec((_TM, _N), lambda i: (jnp.minimum(i, _G - 1), 0)),
            pl.BlockSpec((_N, _D1), lambda i: (0, 0)),
            pl.BlockSpec((_D1, 2 * _D2), lambda i: (0, 0)),
        ],
        out_specs=[
            pl.BlockSpec((_TM, _N), lambda i: (jnp.maximum(i - _G, 0), 0)),
            pl.BlockSpec((_N, _D2), lambda i: (0, 0)),
            pl.BlockSpec((_N, _D2), lambda i: (0, 0)),
        ],
        out_shape=[
            jax.ShapeDtypeStruct((_N, _N), jnp.float32),
            jax.ShapeDtypeStruct((_N, _D2), jnp.float32),
            jax.ShapeDtypeStruct((_N, _D2), jnp.float32),
        ],
        scratch_shapes=[pltpu.VMEM((_N, 2 * _D2), jnp.float32)],
    )(adj, z_in, wcat)

    return (adj_recon, mu, logvar, mu)
